# baseline (speedup 1.0000x reference)
import functools

import jax
import jax.numpy as jnp
from jax import lax
from jax.experimental import pallas as pl
from jax.experimental.pallas import tpu as pltpu
from jax.experimental.pallas import tpu_sc as plsc

_VOCAB = 1000000
_EMBED = 64
_HIDDEN = 256
_OUT = 128
_BATCH = 16384
_BAG = 50

_PAD = 56
_NW = 32
_BAGS_PER_W = _BATCH // _NW
_CHUNK_BAGS = 2
_CHUNK_IDX = _CHUNK_BAGS * _PAD
_CHUNKS = _BAGS_PER_W // _CHUNK_BAGS
_NBUF = 4
_QUADS = _CHUNKS // _NBUF
_OBUF_BAGS = 128
_QUADS_PER_FLUSH = _OBUF_BAGS // (_NBUF * _CHUNK_BAGS)

_sc_mesh = plsc.VectorSubcoreMesh(core_axis_name="c", subcore_axis_name="s")


def _sum_bag(rows_v, row0, k):
    vals = [rows_v[row0 + r, pl.ds(16 * k, 16)] for r in range(_BAG)]
    while len(vals) > 1:
        nxt = [vals[i] + vals[i + 1] for i in range(0, len(vals) - 1, 2)]
        if len(vals) % 2:
            nxt.append(vals[-1])
        vals = nxt
    return vals[0]


@functools.partial(
    pl.kernel,
    mesh=_sc_mesh,
    out_type=jax.ShapeDtypeStruct((_BATCH * _EMBED,), jnp.float32),
    scratch_types=[
        pltpu.VMEM((_CHUNKS * _CHUNK_IDX,), jnp.int32),
        pltpu.VMEM((_CHUNK_IDX, _EMBED), jnp.float32),
        pltpu.VMEM((_CHUNK_IDX, _EMBED), jnp.float32),
        pltpu.VMEM((_CHUNK_IDX, _EMBED), jnp.float32),
        pltpu.VMEM((_CHUNK_IDX, _EMBED), jnp.float32),
        pltpu.VMEM((_OBUF_BAGS * _EMBED,), jnp.float32),
        pltpu.SemaphoreType.DMA,
        pltpu.SemaphoreType.DMA,
        pltpu.SemaphoreType.DMA,
        pltpu.SemaphoreType.DMA,
    ],
    compiler_params=pltpu.CompilerParams(use_tc_tiling_on_sc=False),
)
def _bag_sums(idx_hbm, table_hbm, out_hbm, idx_v, rows0, rows1, rows2, rows3,
              obuf_v, sem0, sem1, sem2, sem3):
    bufs = (rows0, rows1, rows2, rows3)
    sems = (sem0, sem1, sem2, sem3)
    nc = 2
    wid = lax.axis_index("s") * nc + lax.axis_index("c")
    idx_base = wid * (_BAGS_PER_W * _PAD)
    out_base = wid * (_BAGS_PER_W * _EMBED)

    pltpu.sync_copy(idx_hbm.at[pl.ds(idx_base, _CHUNKS * _CHUNK_IDX)], idx_v)

    def gather(c, rows, sem):
        src = table_hbm.at[idx_v.at[pl.ds(c * _CHUNK_IDX, _CHUNK_IDX)]]
        return pltpu.async_copy(src, rows, sem)

    def gather_wait(c, rows, sem):
        src = table_hbm.at[idx_v.at[pl.ds(c * _CHUNK_IDX, _CHUNK_IDX)]]
        pltpu.make_async_copy(src, rows, sem).wait()

    def accum(rows, slot):
        for b in range(_CHUNK_BAGS):
            for k in range(4):
                obuf_v[pl.ds((slot + b) * _EMBED + 16 * k, 16)] = (
                    _sum_bag(rows, b * _PAD, k))

    for j in range(_NBUF - 1):
        gather(j, bufs[j], sems[j])

    def quad_body(q, _):
        c0 = _NBUF * q
        slot = _NBUF * _CHUNK_BAGS * lax.rem(q, _QUADS_PER_FLUSH)
        for j in range(_NBUF):
            c = c0 + j

            @pl.when(c + _NBUF - 1 < _CHUNKS)
            def _(c=c, j=j):
                gather(c + _NBUF - 1, bufs[(j + _NBUF - 1) % _NBUF],
                       sems[(j + _NBUF - 1) % _NBUF])

            gather_wait(c, bufs[j], sems[j])
            accum(bufs[j], slot + _CHUNK_BAGS * j)

        @pl.when(lax.rem(q, _QUADS_PER_FLUSH) == _QUADS_PER_FLUSH - 1)
        def _():
            blk = lax.div(q, _QUADS_PER_FLUSH)
            pltpu.sync_copy(
                obuf_v,
                out_hbm.at[pl.ds(out_base + blk * (_OBUF_BAGS * _EMBED),
                                 _OBUF_BAGS * _EMBED)])
        return 0

    lax.fori_loop(0, _QUADS, quad_body, 0)


def _ffnn_body(x_ref, w1_ref, b1_ref, w2_ref, b2_ref, o_ref):
    x = x_ref[...] * (1.0 / _BAG)
    h = jnp.maximum(
        jnp.dot(x, w1_ref[...], preferred_element_type=jnp.float32)
        + b1_ref[...], 0.0)
    logits = (jnp.dot(h, w2_ref[...], preferred_element_type=jnp.float32)
              + b2_ref[...])
    m = jnp.max(logits, axis=1, keepdims=True)
    shifted = logits - m
    lse = jnp.log(jnp.sum(jnp.exp(shifted), axis=1, keepdims=True))
    o_ref[...] = shifted - lse


_BT = 1024

_ffnn = pl.pallas_call(
    _ffnn_body,
    grid=(_BATCH // _BT,),
    in_specs=[
        pl.BlockSpec((_BT, _EMBED), lambda i: (i, 0)),
        pl.BlockSpec((_EMBED, _HIDDEN), lambda i: (0, 0)),
        pl.BlockSpec((1, _HIDDEN), lambda i: (0, 0)),
        pl.BlockSpec((_HIDDEN, _OUT), lambda i: (0, 0)),
        pl.BlockSpec((1, _OUT), lambda i: (0, 0)),
    ],
    out_specs=pl.BlockSpec((_BT, _OUT), lambda i: (i, 0)),
    out_shape=jax.ShapeDtypeStruct((_BATCH, _OUT), jnp.float32),
)


def kernel(indices, table, W1, b1, W2, b2):
    idx32 = indices.astype(jnp.int32)
    idx_p = jnp.concatenate([idx32, idx32[:, : _PAD - _BAG]], axis=1)
    idx_flat = idx_p.reshape(-1)
    sums = _bag_sums(idx_flat, table).reshape(_BATCH, _EMBED)
    return _ffnn(sums, W1, b1.reshape(1, _HIDDEN), W2, b2.reshape(1, _OUT))


# speedup vs baseline: 1.0381x; 1.0381x over previous
import functools

import jax
import jax.numpy as jnp
from jax import lax
from jax.experimental import pallas as pl
from jax.experimental.pallas import tpu as pltpu
from jax.experimental.pallas import tpu_sc as plsc

_VOCAB = 1000000
_EMBED = 64
_HIDDEN = 256
_OUT = 128
_BATCH = 16384
_BAG = 50

_PAD = 56
_NW = 32
_BAGS_PER_W = _BATCH // _NW
_CHUNK_BAGS = 2
_CHUNK_IDX = _CHUNK_BAGS * _PAD
_CHUNKS = _BAGS_PER_W // _CHUNK_BAGS
_PAIRS = _CHUNKS // 2
_OBUF_BAGS = 128
_PAIRS_PER_FLUSH = _OBUF_BAGS // (2 * _CHUNK_BAGS)

_sc_mesh = plsc.VectorSubcoreMesh(core_axis_name="c", subcore_axis_name="s")


def _sum_bag(rows_v, row0, k):
    vals = [rows_v[row0 + r, pl.ds(16 * k, 16)] for r in range(_BAG)]
    while len(vals) > 1:
        nxt = [vals[i] + vals[i + 1] for i in range(0, len(vals) - 1, 2)]
        if len(vals) % 2:
            nxt.append(vals[-1])
        vals = nxt
    return vals[0]


@functools.partial(
    pl.kernel,
    mesh=_sc_mesh,
    out_type=jax.ShapeDtypeStruct((_BATCH * _EMBED,), jnp.float32),
    scratch_types=[
        pltpu.VMEM((_CHUNKS * _CHUNK_IDX,), jnp.int32),
        pltpu.VMEM((_CHUNK_IDX, _EMBED), jnp.float32),
        pltpu.VMEM((_CHUNK_IDX, _EMBED), jnp.float32),
        pltpu.VMEM((_OBUF_BAGS * _EMBED,), jnp.float32),
        pltpu.SemaphoreType.DMA,
        pltpu.SemaphoreType.DMA,
    ],
    compiler_params=pltpu.CompilerParams(use_tc_tiling_on_sc=False),
)
def _bag_sums(idx_hbm, table_hbm, out_hbm, idx_v, rows0, rows1, obuf_v,
              sem0, sem1):
    nc = 2
    wid = lax.axis_index("s") * nc + lax.axis_index("c")
    idx_base = wid * (_BAGS_PER_W * _PAD)
    out_base = wid * (_BAGS_PER_W * _EMBED)

    pltpu.sync_copy(idx_hbm.at[pl.ds(idx_base, _CHUNKS * _CHUNK_IDX)], idx_v)

    def gather(c, rows, sem):
        src = table_hbm.at[idx_v.at[pl.ds(c * _CHUNK_IDX, _CHUNK_IDX)]]
        return pltpu.async_copy(src, rows, sem)

    def gather_wait(c, rows, sem):
        src = table_hbm.at[idx_v.at[pl.ds(c * _CHUNK_IDX, _CHUNK_IDX)]]
        pltpu.make_async_copy(src, rows, sem).wait()

    def accum(rows, slot):
        for b in range(_CHUNK_BAGS):
            for k in range(4):
                obuf_v[pl.ds((slot + b) * _EMBED + 16 * k, 16)] = (
                    _sum_bag(rows, b * _PAD, k))

    gather(0, rows0, sem0)

    def pair_body(p, _):
        c0 = 2 * p
        gather(c0 + 1, rows1, sem1)
        gather_wait(c0, rows0, sem0)
        slot = 2 * _CHUNK_BAGS * lax.rem(p, _PAIRS_PER_FLUSH)
        accum(rows0, slot)

        @pl.when(p < _PAIRS - 1)
        def _():
            gather(c0 + 2, rows0, sem0)

        gather_wait(c0 + 1, rows1, sem1)
        accum(rows1, slot + _CHUNK_BAGS)

        @pl.when(lax.rem(p, _PAIRS_PER_FLUSH) == _PAIRS_PER_FLUSH - 1)
        def _():
            blk = lax.div(p, _PAIRS_PER_FLUSH)
            pltpu.sync_copy(
                obuf_v,
                out_hbm.at[pl.ds(out_base + blk * (_OBUF_BAGS * _EMBED),
                                 _OBUF_BAGS * _EMBED)])
        return 0

    lax.fori_loop(0, _PAIRS, pair_body, 0)


def _ffnn_body(x_ref, w1_ref, b1_ref, w2_ref, b2_ref, o_ref):
    x = x_ref[...] * (1.0 / _BAG)
    h = jnp.maximum(
        jnp.dot(x, w1_ref[...], preferred_element_type=jnp.float32)
        + b1_ref[...], 0.0)
    logits = (jnp.dot(h, w2_ref[...], preferred_element_type=jnp.float32)
              + b2_ref[...])
    m = jnp.max(logits, axis=1, keepdims=True)
    shifted = logits - m
    lse = jnp.log(jnp.sum(jnp.exp(shifted), axis=1, keepdims=True))
    o_ref[...] = shifted - lse


_BT = 1024

_ffnn = pl.pallas_call(
    _ffnn_body,
    grid=(_BATCH // _BT,),
    in_specs=[
        pl.BlockSpec((_BT, _EMBED), lambda i: (i, 0)),
        pl.BlockSpec((_EMBED, _HIDDEN), lambda i: (0, 0)),
        pl.BlockSpec((1, _HIDDEN), lambda i: (0, 0)),
        pl.BlockSpec((_HIDDEN, _OUT), lambda i: (0, 0)),
        pl.BlockSpec((1, _OUT), lambda i: (0, 0)),
    ],
    out_specs=pl.BlockSpec((_BT, _OUT), lambda i: (i, 0)),
    out_shape=jax.ShapeDtypeStruct((_BATCH, _OUT), jnp.float32),
)


def kernel(indices, table, W1, b1, W2, b2):
    idx32 = indices.astype(jnp.int32)
    idx_p = jnp.concatenate([idx32, idx32[:, : _PAD - _BAG]], axis=1)
    idx_flat = idx_p.reshape(-1)
    sums = _bag_sums(idx_flat, table).reshape(_BATCH, _EMBED)
    return _ffnn(sums, W1, b1.reshape(1, _HIDDEN), W2, b2.reshape(1, _OUT))
